# baseline (speedup 1.0000x reference)
import functools

import jax
import jax.numpy as jnp
from jax import lax
from jax.experimental import pallas as pl
from jax.experimental.pallas import tpu as pltpu
from jax.experimental.pallas import tpu_sc as plsc

IDXW = 128
K = 2
CH = K * IDXW
NCORES = 2


def _build(N, D, V):
    info = plsc.get_sparse_core_info()
    NC, NS = NCORES, info.num_subcores
    NW = NC * NS
    assert N % (NW * CH) == 0
    b_per_w = N // NW
    n_ch = b_per_w // CH
    assert n_ch % 2 == 0 and n_ch >= 4

    mesh = plsc.VectorSubcoreMesh(
        core_axis_name="c", subcore_axis_name="s", num_cores=NC
    )

    @functools.partial(
        pl.kernel,
        mesh=mesh,
        compiler_params=pltpu.CompilerParams(use_tc_tiling_on_sc=False),
        out_type=jax.ShapeDtypeStruct((N, 128), jnp.float32),
        scratch_types=[
            pltpu.VMEM((2, K, IDXW), jnp.int32),
            pltpu.VMEM((2, CH, 128), jnp.float32),
            pltpu.SemaphoreType.DMA,
            pltpu.SemaphoreType.DMA,
        ],
    )
    def emb_kernel(idx_hbm, tbl_hbm, out_hbm, idx_v, rows_v, sg0, sg1):
        wid = lax.axis_index("s") * NC + lax.axis_index("c")
        row0 = wid * (b_per_w // IDXW)
        base = wid * b_per_w
        sg = (sg0, sg1)

        def load_idx(b, c):
            pltpu.sync_copy(idx_hbm.at[pl.ds(row0 + c * K, K)], idx_v.at[b])

        def gathers(b):
            return [
                pltpu.make_async_copy(
                    tbl_hbm.at[idx_v.at[b].at[j]],
                    rows_v.at[b].at[pl.ds(j * IDXW, IDXW)],
                    sg[b],
                )
                for j in range(K)
            ]

        def write_out(b, c):
            pltpu.sync_copy(rows_v.at[b], out_hbm.at[pl.ds(base + c * CH, CH)])

        for b in (0, 1):
            load_idx(b, b)
            for cp in gathers(b):
                cp.start()

        @pl.loop(0, n_ch - 2, step=2)
        def _steady(g):
            for b in (0, 1):
                c = g + b
                for cp in gathers(b):
                    cp.wait()
                write_out(b, c)
                load_idx(b, c + 2)
                for cp in gathers(b):
                    cp.start()

        for b in (0, 1):
            for cp in gathers(b):
                cp.wait()
            write_out(b, n_ch - 2 + b)

    return emb_kernel


def kernel(x, table):
    B, S = x.shape
    V, D = table.shape
    N = B * S
    xf = x.reshape(N).astype(jnp.int32)
    idx2d = (xf >> 1).reshape(N // IDXW, IDXW)
    tblw = table.reshape(V // 2, 2 * D)
    pairs = _build(N, D, V)(idx2d, tblw)
    odd = (xf & 1).astype(jnp.bool_)[:, None]
    out = jnp.where(odd, pairs[:, D:], pairs[:, :D])
    return out.reshape(B, S, D)


# speedup vs baseline: 1.3533x; 1.3533x over previous
import functools

import jax
import jax.numpy as jnp
from jax import lax
from jax.experimental import pallas as pl
from jax.experimental.pallas import tpu as pltpu
from jax.experimental.pallas import tpu_sc as plsc

IDXW = 128
K = 2
CH = K * IDXW
NCORES = 2
PBLK = 8000


def _pad_table(table):
    V, D = table.shape

    def body(t_ref, o_ref):
        o_ref[:, 0:D] = t_ref[...]

    return pl.pallas_call(
        body,
        grid=(V // PBLK,),
        in_specs=[pl.BlockSpec((PBLK, D), lambda i: (i, 0))],
        out_specs=pl.BlockSpec((PBLK, 2 * D), lambda i: (i, 0)),
        out_shape=jax.ShapeDtypeStruct((V, 2 * D), jnp.float32),
    )(table)


def _build(N, D, V):
    info = plsc.get_sparse_core_info()
    NC, NS = NCORES, info.num_subcores
    NW = NC * NS
    assert N % (NW * CH) == 0
    b_per_w = N // NW
    n_ch = b_per_w // CH
    assert n_ch % 2 == 0 and n_ch >= 4

    mesh = plsc.VectorSubcoreMesh(
        core_axis_name="c", subcore_axis_name="s", num_cores=NC
    )

    @functools.partial(
        pl.kernel,
        mesh=mesh,
        out_type=jax.ShapeDtypeStruct((N, 128), jnp.float32),
        scratch_types=[
            pltpu.VMEM((2, CH, 128), jnp.float32),
            pltpu.VMEM((IDXW,), jnp.int32),
            pltpu.VMEM((IDXW,), jnp.int32),
            pltpu.VMEM((IDXW,), jnp.int32),
            pltpu.VMEM((IDXW,), jnp.int32),
            pltpu.SemaphoreType.DMA,
            pltpu.SemaphoreType.DMA,
        ],
    )
    def emb_kernel(idx_hbm, tbl_hbm, out_hbm, rows_v, ix00, ix01, ix10, ix11,
                   sg0, sg1):
        idx_v = ((ix00, ix01), (ix10, ix11))
        wid = lax.axis_index("s") * NC + lax.axis_index("c")
        base = wid * b_per_w
        sg = (sg0, sg1)

        def load_idx(b, c):
            for j in range(K):
                pltpu.sync_copy(
                    idx_hbm.at[pl.ds(base + c * CH + j * IDXW, IDXW)],
                    idx_v[b][j],
                )

        def gathers(b):
            return [
                pltpu.make_async_copy(
                    tbl_hbm.at[idx_v[b][j]],
                    rows_v.at[b].at[pl.ds(j * IDXW, IDXW)],
                    sg[b],
                )
                for j in range(K)
            ]

        def write_out(b, c):
            pltpu.sync_copy(rows_v.at[b], out_hbm.at[pl.ds(base + c * CH, CH)])

        for b in (0, 1):
            load_idx(b, b)
            for cp in gathers(b):
                cp.start()

        @pl.loop(0, n_ch - 2, step=2)
        def _steady(g):
            for b in (0, 1):
                c = g + b
                for cp in gathers(b):
                    cp.wait()
                write_out(b, c)
                load_idx(b, c + 2)
                for cp in gathers(b):
                    cp.start()

        for b in (0, 1):
            for cp in gathers(b):
                cp.wait()
            write_out(b, n_ch - 2 + b)

    return emb_kernel


def kernel(x, table):
    B, S = x.shape
    V, D = table.shape
    N = B * S
    xf = x.reshape(N).astype(jnp.int32)
    ptab = _pad_table(table)
    out = _build(N, D, V)(xf, ptab)
    return out[:, :D].reshape(B, S, D)


# speedup vs baseline: 1.6109x; 1.1903x over previous
import functools

import jax
import jax.numpy as jnp
from jax import lax
from jax.experimental import pallas as pl
from jax.experimental.pallas import tpu as pltpu
from jax.experimental.pallas import tpu_sc as plsc

IDXW = 128
K = 5
CH = K * IDXW
NCORES = 2


def _build(N, D, V):
    info = plsc.get_sparse_core_info()
    NC, NS = NCORES, info.num_subcores
    NW = NC * NS
    assert N % (NW * CH) == 0
    b_per_w = N // NW
    n_ch = b_per_w // CH
    assert n_ch % 2 == 0 and n_ch >= 4

    mesh = plsc.VectorSubcoreMesh(
        core_axis_name="c", subcore_axis_name="s", num_cores=NC
    )

    @functools.partial(
        pl.kernel,
        mesh=mesh,
        compiler_params=pltpu.CompilerParams(use_tc_tiling_on_sc=False),
        out_type=jax.ShapeDtypeStruct((N, 128), jnp.float32),
        scratch_types=[
            pltpu.VMEM((2 * K, IDXW), jnp.int32),
            pltpu.VMEM((2, CH, D), jnp.float32),
            pltpu.SemaphoreType.DMA,
            pltpu.SemaphoreType.DMA,
        ],
    )
    def emb_kernel(idx_hbm, tbl_hbm, out_hbm, idx_v, rows_v, sg0, sg1):
        wid = lax.axis_index("s") * NC + lax.axis_index("c")
        base = wid * b_per_w
        sg = (sg0, sg1)

        def load_idx(b, c):
            for j in range(K):
                pltpu.sync_copy(
                    idx_hbm.at[pl.ds(base + c * CH + j * IDXW, IDXW)],
                    idx_v.at[b * K + j],
                )

        def gathers(b):
            return [
                pltpu.make_async_copy(
                    tbl_hbm.at[idx_v.at[b * K + j]],
                    rows_v.at[b].at[pl.ds(j * IDXW, IDXW)],
                    sg[b],
                )
                for j in range(K)
            ]

        def fire_gathers(b):
            for cp in gathers(b):
                cp.start()

        def wait_gathers(b):
            for cp in gathers(b):
                cp.wait()

        def write_out(b, c):
            pltpu.sync_copy(
                rows_v.at[b],
                out_hbm.at[pl.ds(base + c * CH, CH), pl.ds(0, D)],
            )

        for b in (0, 1):
            load_idx(b, b)
            fire_gathers(b)

        @pl.loop(0, n_ch - 2, step=2)
        def _steady(g):
            for b in (0, 1):
                c = g + b
                wait_gathers(b)
                write_out(b, c)
                load_idx(b, c + 2)
                fire_gathers(b)

        for b in (0, 1):
            wait_gathers(b)
            write_out(b, n_ch - 2 + b)

    return emb_kernel


def kernel(x, table):
    B, S = x.shape
    V, D = table.shape
    N = B * S
    idx1 = x.reshape(N).astype(jnp.int32)
    out = _build(N, D, V)(idx1, table)
    return out[:, :D].reshape(B, S, D)


# speedup vs baseline: 1.7131x; 1.0634x over previous
import functools

import jax
import jax.numpy as jnp
from jax import lax
from jax.experimental import pallas as pl
from jax.experimental.pallas import tpu as pltpu
from jax.experimental.pallas import tpu_sc as plsc

IDXW = 128
K = 5
CH = K * IDXW
NCORES = 2


def _build(N, D, V):
    info = plsc.get_sparse_core_info()
    NC, NS = NCORES, info.num_subcores
    NW = NC * NS
    assert N % (NW * CH) == 0
    b_per_w = N // NW
    n_ch = b_per_w // CH
    assert n_ch % 2 == 0 and n_ch >= 4

    mesh = plsc.VectorSubcoreMesh(
        core_axis_name="c", subcore_axis_name="s", num_cores=NC
    )

    @functools.partial(
        pl.kernel,
        mesh=mesh,
        compiler_params=pltpu.CompilerParams(use_tc_tiling_on_sc=False),
        out_type=jax.ShapeDtypeStruct((N, 128), jnp.float32),
        scratch_types=[
            pltpu.VMEM((2, K, IDXW), jnp.int32),
            pltpu.VMEM((2, CH, D), jnp.float32),
            pltpu.SemaphoreType.DMA,
            pltpu.SemaphoreType.DMA,
        ],
    )
    def emb_kernel(idx_hbm, tbl_hbm, out_hbm, idx_v, rows_v, sg0, sg1):
        wid = lax.axis_index("s") * NC + lax.axis_index("c")
        row0 = wid * (b_per_w // IDXW)
        base = wid * b_per_w
        sg = (sg0, sg1)

        def load_idx(b, c):
            pltpu.sync_copy(idx_hbm.at[pl.ds(row0 + c * K, K)], idx_v.at[b])

        def gathers(b):
            return [
                pltpu.make_async_copy(
                    tbl_hbm.at[idx_v.at[b].at[j]],
                    rows_v.at[b].at[pl.ds(j * IDXW, IDXW)],
                    sg[b],
                )
                for j in range(K)
            ]

        def fire_gathers(b):
            for cp in gathers(b):
                cp.start()

        def wait_gathers(b):
            for cp in gathers(b):
                cp.wait()

        def write_out(b, c):
            pltpu.sync_copy(
                rows_v.at[b],
                out_hbm.at[pl.ds(base + c * CH, CH), pl.ds(0, D)],
            )

        for b in (0, 1):
            load_idx(b, b)
            fire_gathers(b)

        @pl.loop(0, n_ch - 2, step=2)
        def _steady(g):
            for b in (0, 1):
                c = g + b
                wait_gathers(b)
                write_out(b, c)
                load_idx(b, c + 2)
                fire_gathers(b)

        for b in (0, 1):
            wait_gathers(b)
            write_out(b, n_ch - 2 + b)

    return emb_kernel


def kernel(x, table):
    B, S = x.shape
    V, D = table.shape
    N = B * S
    idx2d = x.reshape(N // IDXW, IDXW).astype(jnp.int32)
    out = _build(N, D, V)(idx2d, table)
    return out[:, :D].reshape(B, S, D)
